# baseline (device time: 29286 ns/iter reference)
import functools

import jax
import jax.numpy as jnp
from jax import lax
from jax.experimental import pallas as pl
from jax.experimental.pallas import tpu as pltpu

N_DEV = 8
B, SQ, SKV, D_MODEL = 2, 256, 256, 512
HQ_TOTAL, DH = 32, 64
H_LOC = HQ_TOTAL // N_DEV
D_LOC = H_LOC * DH
BLK = 64

GROUPS = ((0, 176), (176, 176), (352, 160))
MASKS = ((1, 3, 4), (3, 4, 1), (4, 1, 3))
G_MAX = 176
N_STEPS = 3


def kernel(x, Wq, K_ext, V_ext, Wo):
    def body(x_ref, wq_ref, k_ref, v_ref, wo_ref, out_ref,
             acc_ref, wq_loc_ref, wo_loc_ref, sendbuf_ref, comm_ref,
             wq_sem, wo_sem, send_sems, recv_sems):
        my = lax.axis_index("i")
        partners = [jnp.bitwise_xor(my, m) for m in (1, 3, 4)]

        wq_cp = pltpu.make_async_copy(
            wq_ref.at[:, pl.ds(my * D_LOC, D_LOC)], wq_loc_ref, wq_sem)
        wo_cp = pltpu.make_async_copy(
            wo_ref.at[pl.ds(my * D_LOC, D_LOC), :], wo_loc_ref, wo_sem)
        wq_cp.start()
        wo_cp.start()

        barrier_sem = pltpu.get_barrier_semaphore()
        for nbr in partners:
            pl.semaphore_signal(barrier_sem, inc=1, device_id=(nbr,),
                                device_id_type=pl.DeviceIdType.MESH)
        pl.semaphore_wait(barrier_sem, len(partners))

        qb = lax.broadcasted_iota(jnp.int32, (SQ, SKV), 0) // BLK
        kb = lax.broadcasted_iota(jnp.int32, (SQ, SKV), 1) // BLK
        mask = kb <= qb

        import os as _os
        _DIAG = _os.environ.get("KERNEL_DIAG", "")
        if _DIAG == "nocompute":
            wq_cp.wait()
            wo_cp.wait()
            acc_ref[...] = x_ref[...].reshape(B * SQ, D_MODEL) * 0.001
        else:
            wq_cp.wait()
            wo_cp.wait()
        for b in range(B if _DIAG != "nocompute" else 0):
            q = jnp.dot(x_ref[b], wq_loc_ref[...],
                        preferred_element_type=jnp.float32)
            ctx = []
            for h in range(H_LOC):
                qh = q[:, h * DH:(h + 1) * DH]
                kh = k_ref[b, :, h, :]
                vh = v_ref[b, :, h, :]
                s = lax.dot_general(
                    qh, kh, (((1,), (1,)), ((), ())),
                    preferred_element_type=jnp.float32) * 0.125
                s = jnp.where(mask, s, -1e9)
                m = jnp.max(s, axis=-1, keepdims=True)
                w = jnp.exp(s - m)
                w = w / jnp.sum(w, axis=-1, keepdims=True)
                ctx.append(jnp.dot(w, vh, preferred_element_type=jnp.float32))
            acc_ref[pl.ds(b * SQ, SQ), :] = jnp.dot(
                jnp.concatenate(ctx, axis=1), wo_loc_ref[...],
                preferred_element_type=jnp.float32)

        for s in range(N_STEPS if _DIAG != "nocomm" else 0):
            rdmas = []
            for g, (off, ln) in enumerate(GROUPS):
                partner = jnp.bitwise_xor(my, MASKS[g][s])
                sendbuf_ref[g, :ln, :] = acc_ref[pl.ds(off, ln), :].astype(
                    jnp.bfloat16)
                rdma = pltpu.make_async_remote_copy(
                    src_ref=sendbuf_ref.at[g, pl.ds(0, ln)],
                    dst_ref=comm_ref.at[g, s, pl.ds(0, ln)],
                    send_sem=send_sems.at[g, s],
                    recv_sem=recv_sems.at[g, s],
                    device_id=(partner,),
                    device_id_type=pl.DeviceIdType.MESH,
                )
                rdma.start()
                rdmas.append(rdma)
            for g, (off, ln) in enumerate(GROUPS):
                rdmas[g].wait()
                acc_ref[pl.ds(off, ln), :] += comm_ref[g, s, :ln, :].astype(
                    jnp.float32)

        out_ref[...] = acc_ref[...].reshape(B, SQ, D_MODEL)

        @functools.partial(pl.run_scoped,
                           second_barrier=pltpu.SemaphoreType.REGULAR)
        def _(second_barrier):
            for nbr in partners:
                pl.semaphore_signal(second_barrier, inc=1, device_id=(nbr,),
                                    device_id_type=pl.DeviceIdType.MESH)
            pl.semaphore_wait(second_barrier, len(partners))

    return pl.pallas_call(
        body,
        out_shape=jax.ShapeDtypeStruct((B, SQ, D_MODEL), jnp.float32),
        in_specs=[
            pl.BlockSpec(memory_space=pltpu.VMEM),
            pl.BlockSpec(memory_space=pltpu.MemorySpace.HBM),

            pl.BlockSpec(memory_space=pltpu.VMEM),
            pl.BlockSpec(memory_space=pltpu.VMEM),
            pl.BlockSpec(memory_space=pltpu.MemorySpace.HBM),

        ],
        out_specs=pl.BlockSpec(memory_space=pltpu.VMEM),
        scratch_shapes=[
            pltpu.VMEM((B * SQ, D_MODEL), jnp.float32),
            pltpu.VMEM((D_MODEL, D_LOC), jnp.float32),
            pltpu.VMEM((D_LOC, D_MODEL), jnp.float32),
            pltpu.VMEM((3, G_MAX, D_MODEL), jnp.bfloat16),
            pltpu.VMEM((3, N_STEPS, G_MAX, D_MODEL), jnp.bfloat16),
            pltpu.SemaphoreType.DMA,
            pltpu.SemaphoreType.DMA,
            pltpu.SemaphoreType.DMA((3, N_STEPS)),
            pltpu.SemaphoreType.DMA((3, N_STEPS)),
        ],
        compiler_params=pltpu.CompilerParams(collective_id=0),
    )(x, Wq, K_ext, V_ext, Wo)


# device time: 24683 ns/iter; 1.1865x vs baseline; 1.1865x over previous
import functools

import jax
import jax.numpy as jnp
from jax import lax
from jax.experimental import pallas as pl
from jax.experimental.pallas import tpu as pltpu

N_DEV = 8
B, SQ, SKV, D_MODEL = 2, 256, 256, 512
HQ_TOTAL, DH = 32, 64
H_LOC = HQ_TOTAL // N_DEV
D_LOC = H_LOC * DH
BLK = 64

GROUPS = ((0, 176), (176, 176), (352, 160))
MASKS = ((1, 3, 4), (3, 4, 1), (4, 1, 3))
G_MAX = 176
N_STEPS = 3


def kernel(x, Wq, K_ext, V_ext, Wo):
    my_pos = lax.axis_index("i")
    wq_loc = lax.dynamic_slice_in_dim(Wq, my_pos * D_LOC, D_LOC, axis=1)
    wo_loc = lax.dynamic_slice_in_dim(Wo, my_pos * D_LOC, D_LOC, axis=0)
    kt = jnp.transpose(K_ext, (0, 2, 3, 1))
    vt = jnp.transpose(V_ext, (0, 2, 3, 1))

    def body(x_ref, wq_ref, k_ref, v_ref, wo_ref, out_ref,
             acc_ref, sendbuf_ref, comm_ref, send_sems, recv_sems):
        my = lax.axis_index("i")
        partners = [jnp.bitwise_xor(my, m) for m in (1, 3, 4)]

        barrier_sem = pltpu.get_barrier_semaphore()
        for nbr in partners:
            pl.semaphore_signal(barrier_sem, inc=1, device_id=(nbr,),
                                device_id_type=pl.DeviceIdType.MESH)
        pl.semaphore_wait(barrier_sem, len(partners))

        qb = lax.broadcasted_iota(jnp.int32, (SQ, SKV), 0) // BLK
        kb = lax.broadcasted_iota(jnp.int32, (SQ, SKV), 1) // BLK
        mask = kb <= qb

        import os as _os
        _DIAG = _os.environ.get("KERNEL_DIAG", "")
        if _DIAG == "nocompute":
            acc_ref[...] = x_ref[...].reshape(B * SQ, D_MODEL) * 0.001
        for b in range(B if _DIAG != "nocompute" else 0):
            q = jnp.dot(x_ref[b], wq_ref[...],
                        preferred_element_type=jnp.float32)
            ctx = []
            for h in range(H_LOC):
                qh = q[:, h * DH:(h + 1) * DH]
                kh = k_ref[b, h]
                vh = v_ref[b, h]
                s = lax.dot_general(
                    qh, kh, (((1,), (0,)), ((), ())),
                    preferred_element_type=jnp.float32) * 0.125
                s = jnp.where(mask, s, -1e9)
                m = jnp.max(s, axis=-1, keepdims=True)
                w = jnp.exp(s - m)
                w = w / jnp.sum(w, axis=-1, keepdims=True)
                ctx.append(lax.dot_general(
                    w, vh, (((1,), (1,)), ((), ())),
                    preferred_element_type=jnp.float32))
            acc_ref[pl.ds(b * SQ, SQ), :] = jnp.dot(
                jnp.concatenate(ctx, axis=1), wo_ref[...],
                preferred_element_type=jnp.float32)

        for s in range(N_STEPS if _DIAG != "nocomm" else 0):
            rdmas = []
            for g, (off, ln) in enumerate(GROUPS):
                partner = jnp.bitwise_xor(my, MASKS[g][s])
                sendbuf_ref[g, :ln, :] = acc_ref[pl.ds(off, ln), :].astype(
                    jnp.bfloat16)
                rdma = pltpu.make_async_remote_copy(
                    src_ref=sendbuf_ref.at[g, pl.ds(0, ln)],
                    dst_ref=comm_ref.at[g, s, pl.ds(0, ln)],
                    send_sem=send_sems.at[g, s],
                    recv_sem=recv_sems.at[g, s],
                    device_id=(partner,),
                    device_id_type=pl.DeviceIdType.MESH,
                )
                rdma.start()
                rdmas.append(rdma)
            for g, (off, ln) in enumerate(GROUPS):
                rdmas[g].wait()
                acc_ref[pl.ds(off, ln), :] += comm_ref[g, s, :ln, :].astype(
                    jnp.float32)

        out_ref[...] = acc_ref[...].reshape(B, SQ, D_MODEL)

        @functools.partial(pl.run_scoped,
                           second_barrier=pltpu.SemaphoreType.REGULAR)
        def _(second_barrier):
            for nbr in partners:
                pl.semaphore_signal(second_barrier, inc=1, device_id=(nbr,),
                                    device_id_type=pl.DeviceIdType.MESH)
            pl.semaphore_wait(second_barrier, len(partners))

    return pl.pallas_call(
        body,
        out_shape=jax.ShapeDtypeStruct((B, SQ, D_MODEL), jnp.float32),
        in_specs=[pl.BlockSpec(memory_space=pltpu.VMEM)] * 5,
        out_specs=pl.BlockSpec(memory_space=pltpu.VMEM),
        scratch_shapes=[
            pltpu.VMEM((B * SQ, D_MODEL), jnp.float32),
            pltpu.VMEM((3, G_MAX, D_MODEL), jnp.bfloat16),
            pltpu.VMEM((3, N_STEPS, G_MAX, D_MODEL), jnp.bfloat16),
            pltpu.SemaphoreType.DMA((3, N_STEPS)),
            pltpu.SemaphoreType.DMA((3, N_STEPS)),
        ],
        compiler_params=pltpu.CompilerParams(collective_id=0),
    )(x, wq_loc, kt, vt, wo_loc)


# device time: 24499 ns/iter; 1.1954x vs baseline; 1.0075x over previous
import functools

import jax
import jax.numpy as jnp
from jax import lax
from jax.experimental import pallas as pl
from jax.experimental.pallas import tpu as pltpu

N_DEV = 8
B, SQ, SKV, D_MODEL = 2, 256, 256, 512
HQ_TOTAL, DH = 32, 64
H_LOC = HQ_TOTAL // N_DEV
D_LOC = H_LOC * DH
BLK = 64

GROUPS = ((0, 176), (176, 176), (352, 160))
MASKS = ((1, 3, 4), (3, 4, 1), (4, 1, 3))
G_MAX = 176
N_STEPS = 3


def kernel(x, Wq, K_ext, V_ext, Wo):
    my_pos = lax.axis_index("i")
    wq_loc = lax.dynamic_slice_in_dim(Wq, my_pos * D_LOC, D_LOC, axis=1)
    wo_loc = lax.dynamic_slice_in_dim(Wo, my_pos * D_LOC, D_LOC, axis=0)
    kt = jnp.transpose(K_ext, (0, 2, 3, 1))
    vt = jnp.transpose(V_ext, (0, 2, 3, 1))

    def body(x_ref, wq_ref, k_ref, v_ref, wo_ref, out_ref,
             acc_ref, sendbuf_ref, comm_ref, send_sems, recv_sems):
        my = lax.axis_index("i")
        partners = [jnp.bitwise_xor(my, m) for m in (1, 3, 4)]

        barrier_sem = pltpu.get_barrier_semaphore()
        for nbr in partners:
            pl.semaphore_signal(barrier_sem, inc=1, device_id=(nbr,),
                                device_id_type=pl.DeviceIdType.MESH)
        pl.semaphore_wait(barrier_sem, len(partners))

        qb = lax.broadcasted_iota(jnp.int32, (SQ, SKV), 0) // BLK
        kb = lax.broadcasted_iota(jnp.int32, (SQ, SKV), 1) // BLK
        mask = kb <= qb

        import os as _os
        _DIAG = _os.environ.get("KERNEL_DIAG", "")
        if _DIAG == "nocompute":
            acc_ref[...] = x_ref[...].reshape(B * SQ, D_MODEL) * 0.001
        for b in range(B if _DIAG != "nocompute" else 0):
            q = jnp.dot(x_ref[b], wq_ref[...],
                        preferred_element_type=jnp.float32)
            ctx = []
            for h in range(H_LOC):
                qh = q[:, h * DH:(h + 1) * DH]
                kh = k_ref[b, h]
                vh = v_ref[b, h]
                s = lax.dot_general(
                    qh, kh, (((1,), (0,)), ((), ())),
                    preferred_element_type=jnp.float32) * 0.125
                s = jnp.where(mask, s, -1e9)
                m = jnp.max(s, axis=-1, keepdims=True)
                w = jnp.exp(s - m)
                w = w / jnp.sum(w, axis=-1, keepdims=True)
                ctx.append(lax.dot_general(
                    w, vh, (((1,), (1,)), ((), ())),
                    preferred_element_type=jnp.float32))
            acc_ref[pl.ds(b * SQ, SQ), :] = jnp.dot(
                jnp.concatenate(ctx, axis=1), wo_ref[...],
                preferred_element_type=jnp.float32)

        def start(g, s):
            off, ln = GROUPS[g]
            partner = jnp.bitwise_xor(my, MASKS[g][s])
            sendbuf_ref[g, :ln, :] = acc_ref[pl.ds(off, ln), :].astype(
                jnp.bfloat16)
            rdma = pltpu.make_async_remote_copy(
                src_ref=sendbuf_ref.at[g, pl.ds(0, ln)],
                dst_ref=comm_ref.at[g, s, pl.ds(0, ln)],
                send_sem=send_sems.at[g, s],
                recv_sem=recv_sems.at[g, s],
                device_id=(partner,),
                device_id_type=pl.DeviceIdType.MESH,
            )
            rdma.start()
            return rdma

        if _DIAG != "nocomm":
            rdmas = [start(g, 0) for g in range(3)]
            for s in range(N_STEPS):
                for g, (off, ln) in enumerate(GROUPS):
                    rdmas[g].wait()
                    acc_ref[pl.ds(off, ln), :] += comm_ref[
                        g, s, :ln, :].astype(jnp.float32)
                    if s + 1 < N_STEPS:
                        rdmas[g] = start(g, s + 1)

        out_ref[...] = acc_ref[...].reshape(B, SQ, D_MODEL)

        @functools.partial(pl.run_scoped,
                           second_barrier=pltpu.SemaphoreType.REGULAR)
        def _(second_barrier):
            for nbr in partners:
                pl.semaphore_signal(second_barrier, inc=1, device_id=(nbr,),
                                    device_id_type=pl.DeviceIdType.MESH)
            pl.semaphore_wait(second_barrier, len(partners))

    return pl.pallas_call(
        body,
        out_shape=jax.ShapeDtypeStruct((B, SQ, D_MODEL), jnp.float32),
        in_specs=[pl.BlockSpec(memory_space=pltpu.VMEM)] * 5,
        out_specs=pl.BlockSpec(memory_space=pltpu.VMEM),
        scratch_shapes=[
            pltpu.VMEM((B * SQ, D_MODEL), jnp.float32),
            pltpu.VMEM((3, G_MAX, D_MODEL), jnp.bfloat16),
            pltpu.VMEM((3, N_STEPS, G_MAX, D_MODEL), jnp.bfloat16),
            pltpu.SemaphoreType.DMA((3, N_STEPS)),
            pltpu.SemaphoreType.DMA((3, N_STEPS)),
        ],
        compiler_params=pltpu.CompilerParams(collective_id=0),
    )(x, wq_loc, kt, vt, wo_loc)


# device time: 23210 ns/iter; 1.2618x vs baseline; 1.0555x over previous
import functools

import jax
import jax.numpy as jnp
from jax import lax
from jax.experimental import pallas as pl
from jax.experimental.pallas import tpu as pltpu

N_DEV = 8
B, SQ, SKV, D_MODEL = 2, 256, 256, 512
HQ_TOTAL, DH = 32, 64
H_LOC = HQ_TOTAL // N_DEV
D_LOC = H_LOC * DH
BLK = 64

GROUPS = ((0, 176), (176, 176), (352, 160))
MASKS = ((1, 3, 4), (3, 4, 1), (4, 1, 3))
G_MAX = 176
N_STEPS = 3


def kernel(x, Wq, K_ext, V_ext, Wo):
    my_pos = lax.axis_index("i")
    wq_loc = lax.dynamic_slice_in_dim(Wq, my_pos * D_LOC, D_LOC, axis=1)
    wo_loc = lax.dynamic_slice_in_dim(Wo, my_pos * D_LOC, D_LOC, axis=0)
    q_loc = jnp.einsum("bsd,dk->bsk", x, wq_loc,
                       preferred_element_type=jnp.float32).astype(jnp.bfloat16)
    wo_b = wo_loc.astype(jnp.bfloat16)
    kt = jnp.transpose(K_ext, (0, 2, 3, 1)).astype(jnp.bfloat16)
    vt = jnp.transpose(V_ext, (0, 2, 3, 1)).astype(jnp.bfloat16)

    def body(q_ref, k_ref, v_ref, wo_ref, out_ref,
             acc_ref, sendbuf_ref, comm_ref, send_sems, recv_sems):
        my = lax.axis_index("i")
        partners = [jnp.bitwise_xor(my, m) for m in (1, 3, 4)]

        barrier_sem = pltpu.get_barrier_semaphore()
        for nbr in partners:
            pl.semaphore_signal(barrier_sem, inc=1, device_id=(nbr,),
                                device_id_type=pl.DeviceIdType.MESH)
        pl.semaphore_wait(barrier_sem, len(partners))

        qb = lax.broadcasted_iota(jnp.int32, (SQ, SKV), 0) // BLK
        kb = lax.broadcasted_iota(jnp.int32, (SQ, SKV), 1) // BLK
        mask = kb <= qb

        import os as _os
        _DIAG = _os.environ.get("KERNEL_DIAG", "")
        if _DIAG == "nocompute":
            acc_ref[...] = 0.001 * jnp.broadcast_to(
                q_ref[...].reshape(B * SQ, D_LOC).astype(jnp.float32),
                (B * SQ, D_LOC)).repeat(2, axis=1)
        for b in range(B if _DIAG != "nocompute" else 0):
            ctx = []
            for h in range(H_LOC):
                qh = q_ref[b, :, h * DH:(h + 1) * DH]
                kh = k_ref[b, h]
                vh = v_ref[b, h]
                s = lax.dot_general(
                    qh, kh, (((1,), (0,)), ((), ())),
                    preferred_element_type=jnp.float32) * 0.125
                s = jnp.where(mask, s, -1e9)
                m = jnp.max(s, axis=-1, keepdims=True)
                w = jnp.exp(s - m)
                w = (w / jnp.sum(w, axis=-1, keepdims=True)).astype(
                    jnp.bfloat16)
                ctx.append(lax.dot_general(
                    w, vh, (((1,), (1,)), ((), ())),
                    preferred_element_type=jnp.float32,
                    ).astype(jnp.bfloat16))
            acc_ref[pl.ds(b * SQ, SQ), :] = jnp.dot(
                jnp.concatenate(ctx, axis=1), wo_ref[...],
                preferred_element_type=jnp.float32)

        def start(g, s):
            off, ln = GROUPS[g]
            partner = jnp.bitwise_xor(my, MASKS[g][s])
            sendbuf_ref[g, :ln, :] = acc_ref[pl.ds(off, ln), :].astype(
                jnp.bfloat16)
            rdma = pltpu.make_async_remote_copy(
                src_ref=sendbuf_ref.at[g, pl.ds(0, ln)],
                dst_ref=comm_ref.at[g, s, pl.ds(0, ln)],
                send_sem=send_sems.at[g, s],
                recv_sem=recv_sems.at[g, s],
                device_id=(partner,),
                device_id_type=pl.DeviceIdType.MESH,
            )
            rdma.start()
            return rdma

        if _DIAG != "nocomm":
            rdmas = [start(g, 0) for g in range(3)]
            for s in range(N_STEPS):
                for g, (off, ln) in enumerate(GROUPS):
                    rdmas[g].wait()
                    acc_ref[pl.ds(off, ln), :] += comm_ref[
                        g, s, :ln, :].astype(jnp.float32)
                    if s + 1 < N_STEPS:
                        rdmas[g] = start(g, s + 1)

        out_ref[...] = acc_ref[...].reshape(B, SQ, D_MODEL)

        @functools.partial(pl.run_scoped,
                           second_barrier=pltpu.SemaphoreType.REGULAR)
        def _(second_barrier):
            for nbr in partners:
                pl.semaphore_signal(second_barrier, inc=1, device_id=(nbr,),
                                    device_id_type=pl.DeviceIdType.MESH)
            pl.semaphore_wait(second_barrier, len(partners))

    return pl.pallas_call(
        body,
        out_shape=jax.ShapeDtypeStruct((B, SQ, D_MODEL), jnp.float32),
        in_specs=[pl.BlockSpec(memory_space=pltpu.VMEM)] * 4,
        out_specs=pl.BlockSpec(memory_space=pltpu.VMEM),
        scratch_shapes=[
            pltpu.VMEM((B * SQ, D_MODEL), jnp.float32),
            pltpu.VMEM((3, G_MAX, D_MODEL), jnp.bfloat16),
            pltpu.VMEM((3, N_STEPS, G_MAX, D_MODEL), jnp.bfloat16),
            pltpu.SemaphoreType.DMA((3, N_STEPS)),
            pltpu.SemaphoreType.DMA((3, N_STEPS)),
        ],
        compiler_params=pltpu.CompilerParams(collective_id=0),
    )(q_loc, kt, vt, wo_b)
